# baseline (speedup 1.0000x reference)
import functools

import jax
import jax.numpy as jnp
from jax import lax
from jax.experimental import pallas as pl
from jax.experimental.pallas import tpu as pltpu
from jax.experimental.pallas import tpu_sc as plsc

RES = 256
N_PTS = 2097152
LANES = 16
BLK = 2048
NC = 2
NS = 16
NW = NC * NS
PW = N_PTS // NW
N_BLOCKS = PW // BLK
SUBV = BLK // LANES


def _tec_body(coords_hbm, grid_hbm, out_hbm, *s):
    bufs = (s[:23] + (s[46],), s[23:46] + (s[47],))

    wid = lax.axis_index("s") * NC + lax.axis_index("c")

    def stage_a(b, buf):
        (xv, yv, zv, fxv, fyv, fzv) = buf[0:6]
        idx_refs = buf[6:14]
        val_refs = buf[14:22]
        sem = buf[23]
        base = wid * PW + b * BLK
        pltpu.sync_copy(coords_hbm.at[pl.ds(base, BLK)], xv)
        pltpu.sync_copy(coords_hbm.at[pl.ds(N_PTS + base, BLK)], yv)
        pltpu.sync_copy(coords_hbm.at[pl.ds(2 * N_PTS + base, BLK)], zv)

        def pass1(j, c1):
            c = j * LANES
            x = xv[pl.ds(c, LANES)]
            y = yv[pl.ds(c, LANES)]
            z = zv[pl.ds(c, LANES)]
            fscale = jnp.float32(RES - 1)
            ix = (x + 1.0) * 0.5 * fscale
            iy = (y + 1.0) * 0.5 * fscale
            iz = (z + 1.0) * 0.5 * fscale
            xi = ix.astype(jnp.int32)
            yi = iy.astype(jnp.int32)
            zi = iz.astype(jnp.int32)
            fxv[pl.ds(c, LANES)] = ix - xi.astype(jnp.float32)
            fyv[pl.ds(c, LANES)] = iy - yi.astype(jnp.float32)
            fzv[pl.ds(c, LANES)] = iz - zi.astype(jnp.float32)
            fidx = (zi << 16) + (yi << 8) + xi
            idx_refs[0][pl.ds(c, LANES)] = fidx
            idx_refs[1][pl.ds(c, LANES)] = fidx + 1
            idx_refs[2][pl.ds(c, LANES)] = fidx + RES
            idx_refs[3][pl.ds(c, LANES)] = fidx + (RES + 1)
            idx_refs[4][pl.ds(c, LANES)] = fidx + RES * RES
            idx_refs[5][pl.ds(c, LANES)] = fidx + (RES * RES + 1)
            idx_refs[6][pl.ds(c, LANES)] = fidx + (RES * RES + RES)
            idx_refs[7][pl.ds(c, LANES)] = fidx + (RES * RES + RES + 1)
            return c1

        lax.fori_loop(0, SUBV, pass1, 0)
        for idx, val in zip(idx_refs, val_refs):
            pltpu.async_copy(grid_hbm.at[idx], val, sem)

    def drain(buf):
        idx_refs = buf[6:14]
        val_refs = buf[14:22]
        sem = buf[23]
        for idx, val in zip(idx_refs, val_refs):
            pltpu.make_async_copy(grid_hbm.at[idx], val, sem).wait()

    def stage_b(b, buf):
        (fxv, fyv, fzv) = buf[3:6]
        (v000, v001, v010, v011, v100, v101, v110, v111) = buf[14:22]
        outv = buf[22]

        def pass2(j, c2):
            c = j * LANES
            fx = fxv[pl.ds(c, LANES)]
            fy = fyv[pl.ds(c, LANES)]
            fz = fzv[pl.ds(c, LANES)]
            a000 = v000[pl.ds(c, LANES)]
            a001 = v001[pl.ds(c, LANES)]
            a010 = v010[pl.ds(c, LANES)]
            a011 = v011[pl.ds(c, LANES)]
            a100 = v100[pl.ds(c, LANES)]
            a101 = v101[pl.ds(c, LANES)]
            a110 = v110[pl.ds(c, LANES)]
            a111 = v111[pl.ds(c, LANES)]
            b00 = a000 + fx * (a001 - a000)
            b01 = a010 + fx * (a011 - a010)
            b10 = a100 + fx * (a101 - a100)
            b11 = a110 + fx * (a111 - a110)
            c0 = b00 + fy * (b01 - b00)
            c1 = b10 + fy * (b11 - b10)
            outv[pl.ds(c, LANES)] = c0 + fz * (c1 - c0)
            return c2

        lax.fori_loop(0, SUBV, pass2, 0)
        base = wid * PW + b * BLK
        pltpu.sync_copy(outv, out_hbm.at[pl.ds(base, BLK)])

    stage_a(0, bufs[0])

    def pipe(i, carry):
        b0 = 2 * i
        stage_a(b0 + 1, bufs[1])
        drain(bufs[0])
        stage_b(b0, bufs[0])
        stage_a(lax.rem(b0 + 2, N_BLOCKS), bufs[0])
        drain(bufs[1])
        stage_b(b0 + 1, bufs[1])
        return carry

    lax.fori_loop(0, N_BLOCKS // 2, pipe, 0)
    drain(bufs[0])


def _make_sc_call():
    mesh = plsc.VectorSubcoreMesh(core_axis_name="c", subcore_axis_name="s")
    tile_f = pltpu.VMEM((BLK,), jnp.float32)
    tile_i = pltpu.VMEM((BLK,), jnp.int32)
    one_set = [tile_f] * 6 + [tile_i] * 8 + [tile_f] * 8 + [tile_f]
    scratch = one_set * 2 + [pltpu.SemaphoreType.DMA] * 2
    return pl.kernel(
        _tec_body,
        out_type=jax.ShapeDtypeStruct((N_PTS,), jnp.float32),
        mesh=mesh,
        scratch_types=scratch,
    )


_sc_call = _make_sc_call()


@jax.jit
def kernel(xyz_sampled, grid):
    coords = xyz_sampled.reshape(N_PTS, 3).T.reshape(3 * N_PTS)
    gflat = grid.reshape(RES * RES * RES)
    out = _sc_call(coords, gflat)
    return out.reshape(N_PTS)


# speedup vs baseline: 1.9480x; 1.9480x over previous
import functools

import jax
import jax.numpy as jnp
from jax import lax
from jax.experimental import pallas as pl
from jax.experimental.pallas import tpu as pltpu
from jax.experimental.pallas import tpu_sc as plsc

RES = 256
GSZ = RES * RES * RES
N_PTS = 2097152
LANES = 16
BLK = 2048
NC = 2
NS = 16
NW = NC * NS
PW = N_PTS // NW
N_BLOCKS = PW // BLK
SUBV = BLK // LANES


def _tec_body(x_hbm, y_hbm, z_hbm, table_hbm, out_hbm, *s):
    bufs = (s[0:15], s[15:30])
    sems = (s[30], s[31])

    wid = lax.axis_index("s") * NC + lax.axis_index("c")

    def stage_a(b, buf, sem):
        (xv, yv, zv, fxv, fyv, fzv) = buf[0:6]
        idx_refs = buf[6:10]
        val_refs = buf[10:14]
        base = wid * PW + b * BLK
        pltpu.sync_copy(x_hbm.at[pl.ds(base, BLK)], xv)
        pltpu.sync_copy(y_hbm.at[pl.ds(base, BLK)], yv)
        pltpu.sync_copy(z_hbm.at[pl.ds(base, BLK)], zv)

        def pass1(j, c1):
            c = j * LANES
            x = xv[pl.ds(c, LANES)]
            y = yv[pl.ds(c, LANES)]
            z = zv[pl.ds(c, LANES)]
            fscale = jnp.float32(RES - 1)
            ix = (x + 1.0) * 0.5 * fscale
            iy = (y + 1.0) * 0.5 * fscale
            iz = (z + 1.0) * 0.5 * fscale
            xi = ix.astype(jnp.int32)
            yi = iy.astype(jnp.int32)
            zi = iz.astype(jnp.int32)
            fxv[pl.ds(c, LANES)] = ix - xi.astype(jnp.float32)
            fyv[pl.ds(c, LANES)] = iy - yi.astype(jnp.float32)
            fzv[pl.ds(c, LANES)] = iz - zi.astype(jnp.float32)
            fidx = (zi << 16) + (yi << 8) + xi
            idx_refs[0][pl.ds(c, LANES)] = fidx
            idx_refs[1][pl.ds(c, LANES)] = fidx + RES
            idx_refs[2][pl.ds(c, LANES)] = fidx + RES * RES
            idx_refs[3][pl.ds(c, LANES)] = fidx + (RES * RES + RES)
            return c1

        lax.fori_loop(0, SUBV, pass1, 0)
        for idx, val in zip(idx_refs, val_refs):
            pltpu.async_copy(table_hbm.at[idx], val, sem)

    def drain(buf, sem):
        idx_refs = buf[6:10]
        val_refs = buf[10:14]
        for idx, val in zip(idx_refs, val_refs):
            pltpu.make_async_copy(table_hbm.at[idx], val, sem).wait()

    def stage_b(b, buf):
        (fxv, fyv, fzv) = buf[3:6]
        val_refs = buf[10:14]
        outv = buf[14]

        def pass2(j, c2):
            c = j * LANES
            fx = fxv[pl.ds(c, LANES)]
            fy = fyv[pl.ds(c, LANES)]
            fz = fzv[pl.ds(c, LANES)]
            bx = []
            for k in range(4):
                u = val_refs[k][pl.ds(c, LANES)]
                ab = plsc.bitcast(u, jnp.bfloat16)
                v0, v1 = plsc.unpack(ab, format=plsc.PackFormat.INTERLEAVED)
                v0 = v0.astype(jnp.float32)
                v1 = v1.astype(jnp.float32)
                bx.append(v0 + fx * (v1 - v0))
            c0 = bx[0] + fy * (bx[1] - bx[0])
            c1 = bx[2] + fy * (bx[3] - bx[2])
            outv[pl.ds(c, LANES)] = c0 + fz * (c1 - c0)
            return c2

        lax.fori_loop(0, SUBV, pass2, 0)
        base = wid * PW + b * BLK
        pltpu.sync_copy(outv, out_hbm.at[pl.ds(base, BLK)])

    stage_a(0, bufs[0], sems[0])

    def pipe(i, carry):
        b0 = 2 * i
        stage_a(b0 + 1, bufs[1], sems[1])
        drain(bufs[0], sems[0])
        stage_b(b0, bufs[0])
        stage_a(lax.rem(b0 + 2, N_BLOCKS), bufs[0], sems[0])
        drain(bufs[1], sems[1])
        stage_b(b0 + 1, bufs[1])
        return carry

    lax.fori_loop(0, N_BLOCKS // 2, pipe, 0)
    drain(bufs[0], sems[0])


def _make_sc_call():
    mesh = plsc.VectorSubcoreMesh(core_axis_name="c", subcore_axis_name="s")
    tile_f = pltpu.VMEM((BLK,), jnp.float32)
    tile_i = pltpu.VMEM((BLK,), jnp.int32)
    one_set = [tile_f] * 6 + [tile_i] * 4 + [tile_i] * 4 + [tile_f]
    scratch = one_set * 2 + [pltpu.SemaphoreType.DMA] * 2
    return pl.kernel(
        _tec_body,
        out_type=jax.ShapeDtypeStruct((N_PTS,), jnp.float32),
        mesh=mesh,
        scratch_types=scratch,
        compiler_params=pltpu.CompilerParams(needs_layout_passes=False),
    )


_sc_call = _make_sc_call()


@jax.jit
def kernel(xyz_sampled, grid):
    pts = xyz_sampled.reshape(N_PTS, 3)
    g16 = grid.reshape(GSZ).astype(jnp.bfloat16)
    gs16 = jnp.concatenate([g16[1:], g16[:1]])
    lo = lax.bitcast_convert_type(g16, jnp.uint16).astype(jnp.uint32)
    hi = lax.bitcast_convert_type(gs16, jnp.uint16).astype(jnp.uint32)
    table = lax.bitcast_convert_type(lo | (hi << 16), jnp.int32)
    out = _sc_call(pts[:, 0], pts[:, 1], pts[:, 2], table)
    return out.reshape(N_PTS)
